# baseline (speedup 1.0000x reference)
import functools

import jax
import jax.numpy as jnp
from jax import lax
from jax.experimental import pallas as pl
from jax.experimental.pallas import tpu as pltpu
from jax.experimental.pallas import tpu_sc as plsc

_EMB_DIM = 1000000
_OUT_DIM = 64

_NC, _NS = 2, 16
_WINDOW = 128


def _pad_body(t_ref, o_ref):
    t = t_ref[...]
    o_ref[...] = jnp.concatenate([t, jnp.zeros_like(t)], axis=1)


def _pad_table(table):
    v, d = table.shape
    w = 5000
    return pl.pallas_call(
        _pad_body,
        grid=(v // w,),
        in_specs=[pl.BlockSpec((w, d), lambda i: (i, 0))],
        out_specs=pl.BlockSpec((w, 2 * d), lambda i: (i, 0)),
        out_shape=jax.ShapeDtypeStruct((v, 2 * d), table.dtype),
    )(table)


def _sc_gather(table_pad, idx):
    m, w = idx.shape
    d = table_pad.shape[1]
    n = m * w
    mesh = plsc.VectorSubcoreMesh(core_axis_name="core",
                                  subcore_axis_name="subcore")

    @functools.partial(
        pl.kernel,
        out_type=jax.ShapeDtypeStruct((n, d), table_pad.dtype),
        mesh=mesh,
        compiler_params=pltpu.CompilerParams(use_tc_tiling_on_sc=False),
    )
    def gather_kernel(table_hbm, i_hbm, o_hbm):
        def body(i_vmem, o_vmem):
            pltpu.sync_copy(table_hbm.at[i_vmem.at[0]],
                            o_vmem.at[pl.ds(0, _WINDOW)])
            pltpu.sync_copy(table_hbm.at[i_vmem.at[1]],
                            o_vmem.at[pl.ds(_WINDOW, _WINDOW)])

        pltpu.emit_pipeline(
            body,
            grid=(m // 2,),
            in_specs=[pl.BlockSpec((2, _WINDOW),
                                   index_map=lambda i: (i, 0))],
            out_specs=[pl.BlockSpec((2 * _WINDOW, d),
                                    index_map=lambda i: (i, 0))],
            core_axis_name=("core", "subcore"),
            dimension_semantics=(pltpu.PARALLEL,),
        )(i_hbm, o_hbm)

    return gather_kernel(table_pad, idx)


def _ln_t_body(e_ref, g_ref, b_ref, o_ref):
    e = e_ref[...]
    m = jnp.mean(e, axis=1, keepdims=True)
    c = e - m
    var = jnp.mean(c * c, axis=1, keepdims=True)
    o_ref[...] = c * lax.rsqrt(var + 1e-5) * g_ref[...] + b_ref[...]


def _layernorm_t(emb_t, gamma_c, beta_c):
    hist, dp, batch = emb_t.shape
    d = dp // 2
    tblk, bblk = 8, 1024
    return pl.pallas_call(
        _ln_t_body,
        grid=(hist // tblk, batch // bblk),
        in_specs=[
            pl.BlockSpec((tblk, d, bblk), lambda i, j: (i, 0, j)),
            pl.BlockSpec((1, d, bblk), lambda i, j: (0, 0, 0)),
            pl.BlockSpec((1, d, bblk), lambda i, j: (0, 0, 0)),
        ],
        out_specs=pl.BlockSpec((tblk, d, bblk), lambda i, j: (i, 0, j)),
        out_shape=jax.ShapeDtypeStruct((hist, d, batch), emb_t.dtype),
    )(emb_t, gamma_c, beta_c)


def kernel(x, table, gamma, beta):
    batch, hist = x.shape
    row_sums = jnp.sum(x, axis=1, keepdims=True)
    x_norm = x / row_sums * (_EMB_DIM - 1)
    idx = jnp.clip(x_norm, 0, _EMB_DIM - 1).astype(jnp.int32)

    table_pad = _pad_table(table)
    emb_p = _sc_gather(table_pad,
                       idx.reshape(batch * hist // _WINDOW, _WINDOW))
    emb_t = emb_p.reshape(batch, hist, 2 * _OUT_DIM).transpose(1, 2, 0)
    g_c = jnp.broadcast_to(gamma.reshape(1, _OUT_DIM, 1), (1, _OUT_DIM, 1024))
    b_c = jnp.broadcast_to(beta.reshape(1, _OUT_DIM, 1), (1, _OUT_DIM, 1024))
    out_t = _layernorm_t(emb_t, g_c, b_c)
    return out_t.transpose(2, 0, 1)


# speedup vs baseline: 1.1015x; 1.1015x over previous
import functools

import jax
import jax.numpy as jnp
from jax import lax
from jax.experimental import pallas as pl
from jax.experimental.pallas import tpu as pltpu
from jax.experimental.pallas import tpu_sc as plsc

_EMB_DIM = 1000000
_OUT_DIM = 64

_NC, _NS = 2, 16
_WINDOW = 128


def _pad_body(t_ref, o_ref):
    t = t_ref[...]
    o_ref[...] = jnp.concatenate([t, jnp.zeros_like(t)], axis=1)


def _pad_table(table):
    v, d = table.shape
    w = 5000
    return pl.pallas_call(
        _pad_body,
        grid=(v // w,),
        in_specs=[pl.BlockSpec((w, d), lambda i: (i, 0))],
        out_specs=pl.BlockSpec((w, 2 * d), lambda i: (i, 0)),
        out_shape=jax.ShapeDtypeStruct((v, 2 * d), table.dtype),
    )(table)


def _sc_gather(table_pad, idx):
    m, w = idx.shape
    d = table_pad.shape[1]
    n = m * w
    mesh = plsc.VectorSubcoreMesh(core_axis_name="core",
                                  subcore_axis_name="subcore")

    @functools.partial(
        pl.kernel,
        out_type=jax.ShapeDtypeStruct((n, d), table_pad.dtype),
        mesh=mesh,
        compiler_params=pltpu.CompilerParams(use_tc_tiling_on_sc=False),
    )
    def gather_kernel(table_hbm, i_hbm, o_hbm):
        def body(i_vmem, o_vmem):
            pltpu.sync_copy(table_hbm.at[i_vmem.at[0]],
                            o_vmem.at[pl.ds(0, _WINDOW)])
            pltpu.sync_copy(table_hbm.at[i_vmem.at[1]],
                            o_vmem.at[pl.ds(_WINDOW, _WINDOW)])

        pltpu.emit_pipeline(
            body,
            grid=(m // 2,),
            in_specs=[pl.BlockSpec((2, _WINDOW),
                                   index_map=lambda i: (i, 0))],
            out_specs=[pl.BlockSpec((2 * _WINDOW, d),
                                    index_map=lambda i: (i, 0))],
            core_axis_name=("core", "subcore"),
            dimension_semantics=(pltpu.PARALLEL,),
        )(i_hbm, o_hbm)

    return gather_kernel(table_pad, idx)


def _ln_t_body(e_ref, g_ref, b_ref, o_ref):
    e = e_ref[...]
    m = jnp.mean(e, axis=1, keepdims=True)
    c = e - m
    var = jnp.mean(c * c, axis=1, keepdims=True)
    o_ref[...] = c * lax.rsqrt(var + 1e-5) * g_ref[...] + b_ref[...]


def _layernorm_t(emb_t, gamma_c, beta_c):
    hist, dp, batch = emb_t.shape
    d = dp // 2
    tblk, bblk = 8, 1024
    return pl.pallas_call(
        _ln_t_body,
        grid=(hist // tblk, batch // bblk),
        in_specs=[
            pl.BlockSpec((tblk, d, bblk), lambda i, j: (i, 0, j)),
            pl.BlockSpec((1, d, bblk), lambda i, j: (0, 0, 0)),
            pl.BlockSpec((1, d, bblk), lambda i, j: (0, 0, 0)),
        ],
        out_specs=pl.BlockSpec((tblk, d, bblk), lambda i, j: (i, 0, j)),
        out_shape=jax.ShapeDtypeStruct((hist, d, batch), emb_t.dtype),
    )(emb_t, gamma_c, beta_c)


def kernel(x, table, gamma, beta):
    batch, hist = x.shape
    row_sums = jnp.sum(x, axis=1, keepdims=True)
    x_norm = x / row_sums * (_EMB_DIM - 1)
    idx = jnp.clip(x_norm, 0, _EMB_DIM - 1).astype(jnp.int32)

    table_pad = jnp.pad(table, ((0, 0), (0, _OUT_DIM)))
    emb_p = _sc_gather(table_pad,
                       idx.reshape(batch * hist // _WINDOW, _WINDOW))
    emb_t = emb_p.reshape(batch, hist, 2 * _OUT_DIM).transpose(1, 2, 0)
    g_c = jnp.broadcast_to(gamma.reshape(1, _OUT_DIM, 1), (1, _OUT_DIM, 1024))
    b_c = jnp.broadcast_to(beta.reshape(1, _OUT_DIM, 1), (1, _OUT_DIM, 1024))
    out_t = _layernorm_t(emb_t, g_c, b_c)
    return out_t.transpose(2, 0, 1)


# speedup vs baseline: 1.2039x; 1.0929x over previous
import functools

import jax
import jax.numpy as jnp
from jax import lax
from jax.experimental import pallas as pl
from jax.experimental.pallas import tpu as pltpu
from jax.experimental.pallas import tpu_sc as plsc

_EMB_DIM = 1000000
_OUT_DIM = 64

_NC, _NS = 2, 16
_WINDOW = 128


def _pad_body(t_ref, o_ref):
    t = t_ref[...]
    o_ref[...] = jnp.concatenate([t, jnp.zeros_like(t)], axis=1)


def _pad_table(table):
    v, d = table.shape
    w = 5000
    return pl.pallas_call(
        _pad_body,
        grid=(v // w,),
        in_specs=[pl.BlockSpec((w, d), lambda i: (i, 0))],
        out_specs=pl.BlockSpec((w, 2 * d), lambda i: (i, 0)),
        out_shape=jax.ShapeDtypeStruct((v, 2 * d), table.dtype),
    )(table)


def _sc_gather(table_pad, idx):
    m, w = idx.shape
    d = table_pad.shape[1]
    n = m * w
    mesh = plsc.VectorSubcoreMesh(core_axis_name="core",
                                  subcore_axis_name="subcore")

    @functools.partial(
        pl.kernel,
        out_type=jax.ShapeDtypeStruct((n, d), table_pad.dtype),
        mesh=mesh,
        compiler_params=pltpu.CompilerParams(use_tc_tiling_on_sc=False),
    )
    def gather_kernel(table_hbm, i_hbm, o_hbm):
        def body(i_vmem, o_vmem):
            pltpu.sync_copy(table_hbm.at[i_vmem.at[0]],
                            o_vmem.at[pl.ds(0, _WINDOW)])
            pltpu.sync_copy(table_hbm.at[i_vmem.at[1]],
                            o_vmem.at[pl.ds(_WINDOW, _WINDOW)])

        pltpu.emit_pipeline(
            body,
            grid=(m // 2,),
            in_specs=[pl.BlockSpec((2, _WINDOW),
                                   index_map=lambda i: (i, 0))],
            out_specs=[pl.BlockSpec((2 * _WINDOW, d),
                                    index_map=lambda i: (i, 0))],
            core_axis_name=("core", "subcore"),
            dimension_semantics=(pltpu.PARALLEL,),
        )(i_hbm, o_hbm)

    return gather_kernel(table_pad, idx)


def _ln_t_body(e_ref, g_ref, b_ref, o_ref):
    e = e_ref[...]
    m = jnp.mean(e, axis=1, keepdims=True)
    c = e - m
    var = jnp.mean(c * c, axis=1, keepdims=True)
    o_ref[...] = c * lax.rsqrt(var + 1e-5) * g_ref[...] + b_ref[...]


def _layernorm_t(emb_t, gamma_c, beta_c):
    hist, dp, batch = emb_t.shape
    d = _OUT_DIM
    tblk, bblk = 8, 1024
    return pl.pallas_call(
        _ln_t_body,
        grid=(hist // tblk, batch // bblk),
        in_specs=[
            pl.BlockSpec((tblk, d, bblk), lambda i, j: (i, 0, j)),
            pl.BlockSpec((1, d, bblk), lambda i, j: (0, 0, 0)),
            pl.BlockSpec((1, d, bblk), lambda i, j: (0, 0, 0)),
        ],
        out_specs=pl.BlockSpec((tblk, d, bblk), lambda i, j: (i, 0, j)),
        out_shape=jax.ShapeDtypeStruct((hist, d, batch), emb_t.dtype),
    )(emb_t, gamma_c, beta_c)


def kernel(x, table, gamma, beta):
    batch, hist = x.shape
    row_sums = jnp.sum(x, axis=1, keepdims=True)
    x_norm = x / row_sums * (_EMB_DIM - 1)
    idx = jnp.clip(x_norm, 0, _EMB_DIM - 1).astype(jnp.int32)

    table_pad = jnp.pad(table, ((0, 0), (0, _OUT_DIM)))
    emb_p = _sc_gather(table_pad,
                       idx.reshape(batch * hist // _WINDOW, _WINDOW))
    emb_t = emb_p.reshape(batch, hist, 2 * _OUT_DIM)[:, :, :_OUT_DIM]
    emb_t = emb_t.transpose(1, 2, 0)
    g_c = jnp.broadcast_to(gamma.reshape(1, _OUT_DIM, 1), (1, _OUT_DIM, 1024))
    b_c = jnp.broadcast_to(beta.reshape(1, _OUT_DIM, 1), (1, _OUT_DIM, 1024))
    out_t = _layernorm_t(emb_t, g_c, b_c)
    return out_t.transpose(2, 0, 1)


# speedup vs baseline: 1.2320x; 1.0233x over previous
import functools

import jax
import jax.numpy as jnp
from jax import lax
from jax.experimental import pallas as pl
from jax.experimental.pallas import tpu as pltpu
from jax.experimental.pallas import tpu_sc as plsc

_EMB_DIM = 1000000
_OUT_DIM = 64

_NC, _NS = 2, 16
_WINDOW = 128


def _sc_gather(table_pad, idx):
    m, w = idx.shape
    d = table_pad.shape[1]
    n = m * w
    mesh = plsc.VectorSubcoreMesh(core_axis_name="core",
                                  subcore_axis_name="subcore")

    @functools.partial(
        pl.kernel,
        out_type=jax.ShapeDtypeStruct((n, d), table_pad.dtype),
        mesh=mesh,
        compiler_params=pltpu.CompilerParams(use_tc_tiling_on_sc=False),
    )
    def gather_kernel(table_hbm, i_hbm, o_hbm):
        def body(i_vmem, o_vmem):
            pltpu.sync_copy(table_hbm.at[i_vmem.at[0]],
                            o_vmem.at[pl.ds(0, _WINDOW)])
            pltpu.sync_copy(table_hbm.at[i_vmem.at[1]],
                            o_vmem.at[pl.ds(_WINDOW, _WINDOW)])

        pltpu.emit_pipeline(
            body,
            grid=(m // 2,),
            in_specs=[pl.BlockSpec((2, _WINDOW),
                                   index_map=lambda i: (i, 0))],
            out_specs=[pl.BlockSpec((2 * _WINDOW, d),
                                    index_map=lambda i: (i, 0))],
            core_axis_name=("core", "subcore"),
            dimension_semantics=(pltpu.PARALLEL,),
        )(i_hbm, o_hbm)

    return gather_kernel(table_pad, idx)


def _ln_t_body(e_ref, g_ref, b_ref, o_ref):
    e = e_ref[...]
    m = jnp.mean(e, axis=1, keepdims=True)
    c = e - m
    var = jnp.mean(c * c, axis=1, keepdims=True)
    o_ref[...] = c * lax.rsqrt(var + 1e-5) * g_ref[...] + b_ref[...]


def _layernorm_t(emb_t, gamma_c, beta_c):
    hist, dp, batch = emb_t.shape
    d = _OUT_DIM
    tblk, bblk = 8, 2048
    return pl.pallas_call(
        _ln_t_body,
        grid=(hist // tblk, batch // bblk),
        in_specs=[
            pl.BlockSpec((tblk, d, bblk), lambda i, j: (i, 0, j)),
            pl.BlockSpec((1, d, bblk), lambda i, j: (0, 0, 0)),
            pl.BlockSpec((1, d, bblk), lambda i, j: (0, 0, 0)),
        ],
        out_specs=pl.BlockSpec((tblk, d, bblk), lambda i, j: (i, 0, j)),
        out_shape=jax.ShapeDtypeStruct((hist, d, batch), emb_t.dtype),
    )(emb_t, gamma_c, beta_c)


def kernel(x, table, gamma, beta):
    batch, hist = x.shape
    row_sums = jnp.sum(x, axis=1, keepdims=True)
    x_norm = x / row_sums * (_EMB_DIM - 1)
    idx = jnp.clip(x_norm, 0, _EMB_DIM - 1).astype(jnp.int32)

    table_pad = jnp.pad(table, ((0, 0), (0, _OUT_DIM)))
    emb_p = _sc_gather(table_pad,
                       idx.reshape(batch * hist // _WINDOW, _WINDOW))
    emb_t = emb_p.reshape(batch, hist, 2 * _OUT_DIM)[:, :, :_OUT_DIM]
    emb_t = emb_t.transpose(1, 2, 0)
    g_c = jnp.broadcast_to(gamma.reshape(1, _OUT_DIM, 1), (1, _OUT_DIM, 2048))
    b_c = jnp.broadcast_to(beta.reshape(1, _OUT_DIM, 1), (1, _OUT_DIM, 2048))
    out_t = _layernorm_t(emb_t, g_c, b_c)
    return out_t.transpose(2, 0, 1)


# speedup vs baseline: 1.2335x; 1.0012x over previous
import functools

import jax
import jax.numpy as jnp
from jax import lax
from jax.experimental import pallas as pl
from jax.experimental.pallas import tpu as pltpu
from jax.experimental.pallas import tpu_sc as plsc

_EMB_DIM = 1000000
_OUT_DIM = 64

_NC, _NS = 2, 16
_WINDOW = 128


def _sc_gather(table_pad, idx):
    m, w = idx.shape
    d = table_pad.shape[1]
    n = m * w
    mesh = plsc.VectorSubcoreMesh(core_axis_name="core",
                                  subcore_axis_name="subcore")

    @functools.partial(
        pl.kernel,
        out_type=jax.ShapeDtypeStruct((n, d), table_pad.dtype),
        mesh=mesh,
        compiler_params=pltpu.CompilerParams(use_tc_tiling_on_sc=False),
    )
    def gather_kernel(table_hbm, i_hbm, o_hbm):
        def body(i_vmem, o_vmem):
            pltpu.sync_copy(table_hbm.at[i_vmem.at[0]],
                            o_vmem.at[pl.ds(0, _WINDOW)])
            pltpu.sync_copy(table_hbm.at[i_vmem.at[1]],
                            o_vmem.at[pl.ds(_WINDOW, _WINDOW)])

        pltpu.emit_pipeline(
            body,
            grid=(m // 2,),
            in_specs=[pl.BlockSpec((2, _WINDOW),
                                   index_map=lambda i: (i, 0))],
            out_specs=[pl.BlockSpec((2 * _WINDOW, d),
                                    index_map=lambda i: (i, 0))],
            core_axis_name=("core", "subcore"),
            dimension_semantics=(pltpu.PARALLEL,),
        )(i_hbm, o_hbm)

    return gather_kernel(table_pad, idx)


def _ln_t_body(e_ref, g_ref, b_ref, o_ref):
    e = e_ref[...]
    m = jnp.mean(e, axis=1, keepdims=True)
    c = e - m
    var = jnp.mean(c * c, axis=1, keepdims=True)
    o_ref[...] = c * lax.rsqrt(var + 1e-5) * g_ref[...] + b_ref[...]


def _layernorm_t(emb_t, gamma_c, beta_c):
    hist, dp, batch = emb_t.shape
    d = _OUT_DIM
    tblk, bblk = 8, 4096
    return pl.pallas_call(
        _ln_t_body,
        grid=(hist // tblk, batch // bblk),
        in_specs=[
            pl.BlockSpec((tblk, d, bblk), lambda i, j: (i, 0, j)),
            pl.BlockSpec((1, d, bblk), lambda i, j: (0, 0, 0)),
            pl.BlockSpec((1, d, bblk), lambda i, j: (0, 0, 0)),
        ],
        out_specs=pl.BlockSpec((tblk, d, bblk), lambda i, j: (i, 0, j)),
        out_shape=jax.ShapeDtypeStruct((hist, d, batch), emb_t.dtype),
    )(emb_t, gamma_c, beta_c)


def kernel(x, table, gamma, beta):
    batch, hist = x.shape
    row_sums = jnp.sum(x, axis=1, keepdims=True)
    x_norm = x / row_sums * (_EMB_DIM - 1)
    idx = jnp.clip(x_norm, 0, _EMB_DIM - 1).astype(jnp.int32)

    table_pad = jnp.pad(table, ((0, 0), (0, _OUT_DIM)))
    emb_p = _sc_gather(table_pad,
                       idx.reshape(batch * hist // _WINDOW, _WINDOW))
    emb_t = emb_p.reshape(batch, hist, 2 * _OUT_DIM)[:, :, :_OUT_DIM]
    emb_t = emb_t.transpose(1, 2, 0)
    g_c = jnp.broadcast_to(gamma.reshape(1, _OUT_DIM, 1), (1, _OUT_DIM, 4096))
    b_c = jnp.broadcast_to(beta.reshape(1, _OUT_DIM, 1), (1, _OUT_DIM, 4096))
    out_t = _layernorm_t(emb_t, g_c, b_c)
    return out_t.transpose(2, 0, 1)


# speedup vs baseline: 1.2890x; 1.0450x over previous
import functools

import jax
import jax.numpy as jnp
from jax import lax
from jax.experimental import pallas as pl
from jax.experimental.pallas import tpu as pltpu
from jax.experimental.pallas import tpu_sc as plsc

_EMB_DIM = 1000000
_OUT_DIM = 64

_NC, _NS = 2, 16
_WINDOW = 128


def _sc_gather(table_pad, idx):
    m, w = idx.shape
    d = table_pad.shape[1]
    n = m * w
    mesh = plsc.VectorSubcoreMesh(core_axis_name="core",
                                  subcore_axis_name="subcore")

    @functools.partial(
        pl.kernel,
        out_type=jax.ShapeDtypeStruct((n, d), table_pad.dtype),
        mesh=mesh,
        scratch_types=[pltpu.SemaphoreType.DMA, pltpu.SemaphoreType.DMA],
        compiler_params=pltpu.CompilerParams(use_tc_tiling_on_sc=False),
    )
    def gather_kernel(table_hbm, i_hbm, o_hbm, sem_a, sem_b):
        def body(i_vmem, o_vmem):
            ca = pltpu.async_copy(table_hbm.at[i_vmem.at[0]],
                                  o_vmem.at[pl.ds(0, _WINDOW)], sem_a)
            cb = pltpu.async_copy(table_hbm.at[i_vmem.at[1]],
                                  o_vmem.at[pl.ds(_WINDOW, _WINDOW)], sem_b)
            ca.wait()
            cb.wait()

        pltpu.emit_pipeline(
            body,
            grid=(m // 2,),
            in_specs=[pl.BlockSpec((2, _WINDOW),
                                   index_map=lambda i: (i, 0))],
            out_specs=[pl.BlockSpec((2 * _WINDOW, d),
                                    index_map=lambda i: (i, 0))],
            core_axis_name=("core", "subcore"),
            dimension_semantics=(pltpu.PARALLEL,),
        )(i_hbm, o_hbm)

    return gather_kernel(table_pad, idx)


def _ln_t_body(e_ref, g_ref, b_ref, o_ref):
    e = e_ref[...]
    m = jnp.mean(e, axis=1, keepdims=True)
    c = e - m
    var = jnp.mean(c * c, axis=1, keepdims=True)
    o_ref[...] = c * lax.rsqrt(var + 1e-5) * g_ref[...] + b_ref[...]


def _layernorm_t(emb_t, gamma_c, beta_c):
    hist, dp, batch = emb_t.shape
    d = _OUT_DIM
    tblk, bblk = 8, 4096
    return pl.pallas_call(
        _ln_t_body,
        grid=(hist // tblk, batch // bblk),
        in_specs=[
            pl.BlockSpec((tblk, d, bblk), lambda i, j: (i, 0, j)),
            pl.BlockSpec((1, d, bblk), lambda i, j: (0, 0, 0)),
            pl.BlockSpec((1, d, bblk), lambda i, j: (0, 0, 0)),
        ],
        out_specs=pl.BlockSpec((tblk, d, bblk), lambda i, j: (i, 0, j)),
        out_shape=jax.ShapeDtypeStruct((hist, d, batch), emb_t.dtype),
    )(emb_t, gamma_c, beta_c)


def kernel(x, table, gamma, beta):
    batch, hist = x.shape
    row_sums = jnp.sum(x, axis=1, keepdims=True)
    x_norm = x / row_sums * (_EMB_DIM - 1)
    idx = jnp.clip(x_norm, 0, _EMB_DIM - 1).astype(jnp.int32)

    table_pad = jnp.pad(table, ((0, 0), (0, _OUT_DIM)))
    emb_p = _sc_gather(table_pad,
                       idx.reshape(batch * hist // _WINDOW, _WINDOW))
    emb_t = emb_p.reshape(batch, hist, 2 * _OUT_DIM)[:, :, :_OUT_DIM]
    emb_t = emb_t.transpose(1, 2, 0)
    g_c = jnp.broadcast_to(gamma.reshape(1, _OUT_DIM, 1), (1, _OUT_DIM, 4096))
    b_c = jnp.broadcast_to(beta.reshape(1, _OUT_DIM, 1), (1, _OUT_DIM, 4096))
    out_t = _layernorm_t(emb_t, g_c, b_c)
    return out_t.transpose(2, 0, 1)


# speedup vs baseline: 1.2904x; 1.0011x over previous
import functools

import jax
import jax.numpy as jnp
from jax import lax
from jax.experimental import pallas as pl
from jax.experimental.pallas import tpu as pltpu
from jax.experimental.pallas import tpu_sc as plsc

_EMB_DIM = 1000000
_OUT_DIM = 64

_NC, _NS = 2, 16
_WINDOW = 128


def _sc_gather(table_pad, idx):
    m, w = idx.shape
    d = table_pad.shape[1]
    n = m * w
    mesh = plsc.VectorSubcoreMesh(core_axis_name="core",
                                  subcore_axis_name="subcore")

    @functools.partial(
        pl.kernel,
        out_type=jax.ShapeDtypeStruct((n, d), table_pad.dtype),
        mesh=mesh,
        scratch_types=[pltpu.SemaphoreType.DMA, pltpu.SemaphoreType.DMA],
        compiler_params=pltpu.CompilerParams(use_tc_tiling_on_sc=False),
    )
    def gather_kernel(table_hbm, i_hbm, o_hbm, sem_a, sem_b):
        def body(i_vmem, o_vmem):
            ca = pltpu.async_copy(table_hbm.at[i_vmem.at[0]],
                                  o_vmem.at[pl.ds(0, _WINDOW)], sem_a)
            cb = pltpu.async_copy(table_hbm.at[i_vmem.at[1]],
                                  o_vmem.at[pl.ds(_WINDOW, _WINDOW)], sem_b)
            ca.wait()
            cb.wait()

        pltpu.emit_pipeline(
            body,
            grid=(m // 2,),
            in_specs=[pl.BlockSpec((2, _WINDOW),
                                   index_map=lambda i: (i, 0))],
            out_specs=[pl.BlockSpec((2 * _WINDOW, d),
                                    index_map=lambda i: (i, 0))],
            core_axis_name=("core", "subcore"),
            dimension_semantics=(pltpu.PARALLEL,),
        )(i_hbm, o_hbm)

    return gather_kernel(table_pad, idx)


def _ln_t_body(e_ref, g_ref, b_ref, o_ref):
    e = e_ref[...]
    m = jnp.mean(e, axis=1, keepdims=True)
    c = e - m
    var = jnp.mean(c * c, axis=1, keepdims=True)
    o_ref[...] = c * lax.rsqrt(var + 1e-5) * g_ref[...] + b_ref[...]


def _layernorm_t(emb_t, gamma_c, beta_c):
    hist, d, batch = emb_t.shape
    tblk, bblk = 8, 4096
    return pl.pallas_call(
        _ln_t_body,
        grid=(hist // tblk, batch // bblk),
        in_specs=[
            pl.BlockSpec((tblk, d, bblk), lambda i, j: (i, 0, j)),
            pl.BlockSpec((1, d, bblk), lambda i, j: (0, 0, 0)),
            pl.BlockSpec((1, d, bblk), lambda i, j: (0, 0, 0)),
        ],
        out_specs=pl.BlockSpec((tblk, d, bblk), lambda i, j: (i, 0, j)),
        out_shape=jax.ShapeDtypeStruct((hist, d, batch), emb_t.dtype),
    )(emb_t, gamma_c, beta_c)


def kernel(x, table, gamma, beta):
    batch, hist = x.shape
    row_sums = jnp.sum(x, axis=1, keepdims=True)
    x_norm = x / row_sums * (_EMB_DIM - 1)
    idx = jnp.clip(x_norm, 0, _EMB_DIM - 1).astype(jnp.int32)

    table_pad = jnp.pad(table, ((0, 0), (0, _OUT_DIM)))
    emb_p = _sc_gather(table_pad,
                       idx.reshape(batch * hist // _WINDOW, _WINDOW))
    emb_t = emb_p.reshape(batch, hist, 2 * _OUT_DIM)[:, :, :_OUT_DIM]
    emb_t = emb_t.transpose(1, 2, 0)
    g_c = jnp.broadcast_to(gamma.reshape(1, _OUT_DIM, 1), (1, _OUT_DIM, 4096))
    b_c = jnp.broadcast_to(beta.reshape(1, _OUT_DIM, 1), (1, _OUT_DIM, 4096))
    out_t = _layernorm_t(emb_t, g_c, b_c)
    return out_t.transpose(2, 0, 1)
